# Initial kernel scaffold; baseline (speedup 1.0000x reference)
import jax
import jax.numpy as jnp
from jax.experimental import pallas as pl


def kernel(test_feature, train_features, train_labels):
    raise NotImplementedError("write your pallas kernel here")


# speedup vs baseline: 1.0647x; 1.0647x over previous
import jax
import jax.numpy as jnp
from jax.experimental import pallas as pl

K = 20
PATCH = 16
NROWS = 14
NUM_PATCHES = NROWS * NROWS
D = 384
N_TRAIN = 500
BS = 16
BK = BS * K


def _body(test_ref, feat_ref, lab_ref, out_ref, dist_ref):
    t = test_ref[0]
    f = feat_ref[0]
    sim = jnp.dot(t, f, preferred_element_type=jnp.float32)

    iota = jax.lax.broadcasted_iota(jnp.int32, (BS, N_TRAIN), 1)
    dists = []
    idxs = []
    for _ in range(K):
        m = jnp.max(sim, axis=1, keepdims=True)
        amin = jnp.min(jnp.where(sim == m, iota, N_TRAIN), axis=1,
                       keepdims=True)
        dists.append(m)
        idxs.append(amin)
        sim = jnp.where(iota == amin, -jnp.inf, sim)
    dist = jnp.concatenate(dists, axis=1)
    idx = jnp.concatenate(idxs, axis=1)
    dist_ref[0] = dist

    idx_t = jnp.concatenate([idx] * BS, axis=1)
    lane = jax.lax.broadcasted_iota(jnp.int32, (BS, BK), 1)
    row = jax.lax.broadcasted_iota(jnp.int32, (BS, BK), 0)
    idx_row = jnp.max(jnp.where(lane // K == row, idx_t, -1), axis=0,
                      keepdims=True)

    iota_n = jax.lax.broadcasted_iota(jnp.int32, (N_TRAIN, BK), 0)
    oh = (iota_n == idx_row).astype(jnp.bfloat16)
    labf = lab_ref[0].astype(jnp.bfloat16)
    ret = jax.lax.dot_general(oh, labf, (((0,), (1,)), ((), ())),
                              preferred_element_type=jnp.float32)
    ret = ret.astype(jnp.int32)
    for i in range(PATCH):
        out_ref[:, 0, i, 0, 0, :] = ret[:, i * PATCH:(i + 1) * PATCH]


def kernel(test_feature, train_features, train_labels):
    test2 = jnp.transpose(test_feature, (1, 0, 2))

    out6, dist3 = pl.pallas_call(
        _body,
        grid=(NUM_PATCHES,),
        in_specs=[
            pl.BlockSpec((1, BS, D), lambda p: (p, 0, 0)),
            pl.BlockSpec((1, D, N_TRAIN), lambda p: (p, 0, 0)),
            pl.BlockSpec((1, PATCH * PATCH, N_TRAIN), lambda p: (p, 0, 0)),
        ],
        out_specs=[
            pl.BlockSpec((BK, 1, PATCH, 1, 1, PATCH),
                         lambda p: (0, p // NROWS, 0, p % NROWS, 0, 0)),
            pl.BlockSpec((1, BS, K), lambda p: (p, 0, 0)),
        ],
        out_shape=[
            jax.ShapeDtypeStruct((BK, NROWS, PATCH, NROWS, 1, PATCH),
                                 jnp.int32),
            jax.ShapeDtypeStruct((NUM_PATCHES, BS, K), jnp.float32),
        ],
    )(test2, train_features, train_labels)

    grids = out6.reshape(BS, K, NROWS * PATCH, NROWS * PATCH)
    distances = jnp.transpose(dist3, (1, 0, 2))
    return grids, distances


# speedup vs baseline: 2.4732x; 2.3229x over previous
import functools

import jax
import jax.numpy as jnp
from jax import lax
from jax.experimental import pallas as pl
from jax.experimental.pallas import tpu as pltpu
from jax.experimental.pallas import tpu_sc as plsc

K = 20
PATCH = 16
NROWS = 14
NUM_PATCHES = NROWS * NROWS
D = 384
N_TRAIN = 500
BS = 16
HW = NROWS * PATCH

NCHUNK = 4
CROWS = (PATCH * PATCH) // NCHUNK
CPIX = CROWS // PATCH

NW = 32
PPW = 7


def _topk_body(test_ref, feat_ref, idx_ref, dist_ref):
    t = test_ref[0]
    f = feat_ref[0]
    sim = jnp.dot(t, f, preferred_element_type=jnp.float32)

    iota = lax.broadcasted_iota(jnp.int32, (BS, N_TRAIN), 1)
    dists = []
    idxs = []
    for _ in range(K):
        m = jnp.max(sim, axis=1, keepdims=True)
        amin = jnp.min(jnp.where(sim == m, iota, N_TRAIN), axis=1,
                       keepdims=True)
        dists.append(m)
        idxs.append(amin)
        sim = jnp.where(iota == amin, -jnp.inf, sim)
    dist_ref[0] = jnp.concatenate(dists, axis=1)
    idx_ref[0] = jnp.concatenate(idxs, axis=1)


def _topk(test_feature, train_features):
    test2 = jnp.transpose(test_feature, (1, 0, 2))
    return pl.pallas_call(
        _topk_body,
        grid=(NUM_PATCHES,),
        in_specs=[
            pl.BlockSpec((1, BS, D), lambda p: (p, 0, 0)),
            pl.BlockSpec((1, D, N_TRAIN), lambda p: (p, 0, 0)),
        ],
        out_specs=[
            pl.BlockSpec((1, BS, K), lambda p: (p, 0, 0)),
            pl.BlockSpec((1, BS, K), lambda p: (p, 0, 0)),
        ],
        out_shape=[
            jax.ShapeDtypeStruct((NUM_PATCHES, BS, K), jnp.int32),
            jax.ShapeDtypeStruct((NUM_PATCHES, BS, K), jnp.float32),
        ],
    )(test2, train_features)


def _gather_body(lab_hbm, idx_hbm, out_hbm, labq, idxv, stage, sem_out):
    wid = lax.axis_index("s") * 2 + lax.axis_index("c")
    lane = lax.broadcasted_iota(jnp.int32, (PATCH,), 0)

    def per_patch(t, carry):
        p = t * NW + wid

        @pl.when(p < NUM_PATCHES)
        def _do():
            r = p // NROWS
            c = p % NROWS
            pltpu.sync_copy(idx_hbm.at[p], idxv)
            for q in range(NCHUNK):
                pltpu.sync_copy(lab_hbm.at[p, pl.ds(q * CROWS, CROWS), :],
                                labq)

                def per_b(b, _c):
                    v0 = idxv[b, pl.ds(0, 16)]
                    v1 = idxv[b, pl.ds(K - 16, 16)]
                    for k in range(K):
                        val = v0[k] if k < 16 else v1[k - (K - 16)]
                        col = jnp.full((PATCH,), val, jnp.int32)
                        for il in range(CPIX):
                            vec = plsc.load_gather(
                                labq, [lane + il * PATCH, col])
                            stage[b, k, il] = vec
                    return _c

                lax.fori_loop(0, BS, per_b, 0)
                copies = []
                for b in range(BS):
                    cp = pltpu.make_async_copy(
                        stage.at[b],
                        out_hbm.at[b, :, pl.ds(r * PATCH + q * CPIX, CPIX),
                                   pl.ds(c * PATCH, PATCH)],
                        sem_out)
                    cp.start()
                    copies.append(cp)
                for cp in copies:
                    cp.wait()
        return carry

    lax.fori_loop(0, PPW, per_patch, 0)


def _gather(train_labels, idx3):
    kern = functools.partial(
        pl.kernel,
        out_type=jax.ShapeDtypeStruct((BS, K, HW, HW), jnp.int32),
        mesh=plsc.VectorSubcoreMesh(core_axis_name="c", subcore_axis_name="s"),
        scratch_types=[
            pltpu.VMEM((CROWS, N_TRAIN), jnp.int32),
            pltpu.VMEM((BS, K), jnp.int32),
            pltpu.VMEM((BS, K, CPIX, PATCH), jnp.int32),
            pltpu.SemaphoreType.DMA,
        ],
        compiler_params=pltpu.CompilerParams(use_tc_tiling_on_sc=False,
                                             needs_layout_passes=False),
    )(_gather_body)
    return kern(train_labels, idx3)


def kernel(test_feature, train_features, train_labels):
    idx3, dist3 = _topk(test_feature, train_features)
    grids = _gather(train_labels, idx3)
    distances = jnp.transpose(dist3, (1, 0, 2))
    return grids, distances


# speedup vs baseline: 4.7435x; 1.9179x over previous
import functools

import jax
import jax.numpy as jnp
from jax import lax
from jax.experimental import pallas as pl
from jax.experimental.pallas import tpu as pltpu
from jax.experimental.pallas import tpu_sc as plsc

K = 20
PATCH = 16
NROWS = 14
NUM_PATCHES = NROWS * NROWS
D = 384
N_TRAIN = 500
BS = 16
HW = NROWS * PATCH

NCHUNK = 4
CROWS = (PATCH * PATCH) // NCHUNK
CPIX = CROWS // PATCH

NW = 32
PPW = 7


PB = 8
NSTEP = (NUM_PATCHES + PB - 1) // PB
ROWS = PB * BS


def _topk_body(test_ref, feat_ref, idx_ref, dist_ref):
    sims = []
    for j in range(PB):
        t = test_ref[:, j, :]
        f = feat_ref[j]
        sims.append(jnp.dot(t, f, preferred_element_type=jnp.float32))
    sim = jnp.concatenate(sims, axis=0)

    iota = lax.broadcasted_iota(jnp.int32, (ROWS, N_TRAIN), 1)
    dists = []
    idxs = []
    for _ in range(K):
        m = jnp.max(sim, axis=1, keepdims=True)
        amin = jnp.min(jnp.where(sim == m, iota, N_TRAIN), axis=1,
                       keepdims=True)
        dists.append(m)
        idxs.append(amin)
        sim = jnp.where(iota == amin, -jnp.inf, sim)
    dist_ref[0] = jnp.concatenate(dists, axis=1)
    idx_ref[0] = jnp.concatenate(idxs, axis=1)


def _topk(test_feature, train_features):
    idxo, disto = pl.pallas_call(
        _topk_body,
        grid=(NSTEP,),
        in_specs=[
            pl.BlockSpec((BS, PB, D), lambda p: (0, p, 0)),
            pl.BlockSpec((PB, D, N_TRAIN), lambda p: (p, 0, 0)),
        ],
        out_specs=[
            pl.BlockSpec((1, ROWS, K), lambda p: (p, 0, 0)),
            pl.BlockSpec((1, ROWS, K), lambda p: (p, 0, 0)),
        ],
        out_shape=[
            jax.ShapeDtypeStruct((NSTEP, ROWS, K), jnp.int32),
            jax.ShapeDtypeStruct((NSTEP, ROWS, K), jnp.float32),
        ],
    )(test_feature, train_features)
    idx3 = idxo.reshape(NSTEP * PB, BS, K)[:NUM_PATCHES]
    dist3 = disto.reshape(NSTEP * PB, BS, K)[:NUM_PATCHES]
    return idx3, dist3


def _gather_body(lab_hbm, idx_hbm, out_hbm, labq, idxv, stage, sem_out):
    wid = lax.axis_index("s") * 2 + lax.axis_index("c")
    lane = lax.broadcasted_iota(jnp.int32, (PATCH,), 0)

    def per_patch(t, carry):
        p = t * NW + wid

        @pl.when(p < NUM_PATCHES)
        def _do():
            r = p // NROWS
            c = p % NROWS
            pltpu.sync_copy(idx_hbm.at[p], idxv)
            for q in range(NCHUNK):
                pltpu.sync_copy(lab_hbm.at[p, pl.ds(q * CROWS, CROWS), :],
                                labq)

                def per_b(b, _c):
                    v0 = idxv[b, pl.ds(0, 16)]
                    v1 = idxv[b, pl.ds(K - 16, 16)]
                    for k in range(K):
                        val = v0[k] if k < 16 else v1[k - (K - 16)]
                        col = jnp.full((PATCH,), val, jnp.int32)
                        for il in range(CPIX):
                            vec = plsc.load_gather(
                                labq, [lane + il * PATCH, col])
                            stage[b, k, il] = vec
                    return _c

                lax.fori_loop(0, BS, per_b, 0)
                copies = []
                for b in range(BS):
                    cp = pltpu.make_async_copy(
                        stage.at[b],
                        out_hbm.at[b, :, pl.ds(r * PATCH + q * CPIX, CPIX),
                                   pl.ds(c * PATCH, PATCH)],
                        sem_out)
                    cp.start()
                    copies.append(cp)
                for cp in copies:
                    cp.wait()
        return carry

    lax.fori_loop(0, PPW, per_patch, 0)


def _gather(train_labels, idx3):
    kern = functools.partial(
        pl.kernel,
        out_type=jax.ShapeDtypeStruct((BS, K, HW, HW), jnp.int32),
        mesh=plsc.VectorSubcoreMesh(core_axis_name="c", subcore_axis_name="s"),
        scratch_types=[
            pltpu.VMEM((CROWS, N_TRAIN), jnp.int32),
            pltpu.VMEM((BS, K), jnp.int32),
            pltpu.VMEM((BS, K, CPIX, PATCH), jnp.int32),
            pltpu.SemaphoreType.DMA,
        ],
        compiler_params=pltpu.CompilerParams(use_tc_tiling_on_sc=False,
                                             needs_layout_passes=False),
    )(_gather_body)
    return kern(train_labels, idx3)


def kernel(test_feature, train_features, train_labels):
    idx3, dist3 = _topk(test_feature, train_features)
    grids = _gather(train_labels, idx3)
    distances = jnp.transpose(dist3, (1, 0, 2))
    return grids, distances


# speedup vs baseline: 5.0189x; 1.0581x over previous
import functools

import jax
import jax.numpy as jnp
from jax import lax
from jax.experimental import pallas as pl
from jax.experimental.pallas import tpu as pltpu
from jax.experimental.pallas import tpu_sc as plsc

K = 20
PATCH = 16
NROWS = 14
NUM_PATCHES = NROWS * NROWS
D = 384
N_TRAIN = 500
BS = 16
HW = NROWS * PATCH

NCHUNK = 4
CROWS = (PATCH * PATCH) // NCHUNK
CPIX = CROWS // PATCH

NW = 32
PPW = 7


PB = 8
NSTEP = (NUM_PATCHES + PB - 1) // PB
ROWS = PB * BS


def _topk_body(test_ref, feat_ref, idx_ref, dist_ref):
    sims = []
    for j in range(PB):
        t = test_ref[:, j, :]
        f = feat_ref[j]
        sims.append(jnp.dot(t, f, preferred_element_type=jnp.float32))
    sim = jnp.concatenate(sims, axis=0)

    iota = lax.broadcasted_iota(jnp.int32, (ROWS, N_TRAIN), 1)
    dists = []
    idxs = []
    for _ in range(K):
        m = jnp.max(sim, axis=1, keepdims=True)
        amin = jnp.min(jnp.where(sim == m, iota, N_TRAIN), axis=1,
                       keepdims=True)
        dists.append(m)
        idxs.append(amin)
        sim = jnp.where(iota == amin, -jnp.inf, sim)
    dist_ref[0] = jnp.concatenate(dists, axis=1)
    idx_ref[0] = jnp.concatenate(idxs, axis=1)


def _topk(test_feature, train_features):
    idxo, disto = pl.pallas_call(
        _topk_body,
        grid=(NSTEP,),
        in_specs=[
            pl.BlockSpec((BS, PB, D), lambda p: (0, p, 0)),
            pl.BlockSpec((PB, D, N_TRAIN), lambda p: (p, 0, 0)),
        ],
        out_specs=[
            pl.BlockSpec((1, ROWS, K), lambda p: (p, 0, 0)),
            pl.BlockSpec((1, ROWS, K), lambda p: (p, 0, 0)),
        ],
        out_shape=[
            jax.ShapeDtypeStruct((NSTEP, ROWS, K), jnp.int32),
            jax.ShapeDtypeStruct((NSTEP, ROWS, K), jnp.float32),
        ],
    )(test_feature, train_features)
    idx3 = idxo.reshape(NSTEP * PB, BS, K)[:NUM_PATCHES]
    dist3 = disto.reshape(NSTEP * PB, BS, K)[:NUM_PATCHES]
    return idx3, dist3


def _gather_body(lab_hbm, idx_hbm, out_hbm, labq, idxv, stage,
                 sem_in, sem_out0, sem_out1):
    wid = lax.axis_index("s") * 2 + lax.axis_index("c")
    lane = lax.broadcasted_iota(jnp.int32, (PATCH,), 0)
    sem_out = (sem_out0, sem_out1)

    def per_patch(t, carry):
        p = t * NW + wid

        @pl.when(p < NUM_PATCHES)
        def _do():
            r = p // NROWS
            c = p % NROWS
            pltpu.sync_copy(idx_hbm.at[p], idxv)

            def in_copy(q):
                cp = pltpu.make_async_copy(
                    lab_hbm.at[p, pl.ds(q * CROWS, CROWS), :],
                    labq.at[q % 2], sem_in)
                cp.start()
                return cp

            def out_copies(q):
                par = q % 2
                cps = []
                for b in range(BS):
                    cp = pltpu.make_async_copy(
                        stage.at[par, b],
                        out_hbm.at[b, :, pl.ds(r * PATCH + q * CPIX, CPIX),
                                   pl.ds(c * PATCH, PATCH)],
                        sem_out[par])
                    cp.start()
                    cps.append(cp)
                return cps

            pending_in = in_copy(0)
            pending_out = {0: [], 1: []}
            for q in range(NCHUNK):
                par = q % 2
                pending_in.wait()
                if q + 1 < NCHUNK:
                    pending_in = in_copy(q + 1)
                for cp in pending_out[par]:
                    cp.wait()

                def per_b(b, _c):
                    v0 = idxv[b, pl.ds(0, 16)]
                    v1 = idxv[b, pl.ds(K - 16, 16)]
                    for k in range(K):
                        val = v0[k] if k < 16 else v1[k - (K - 16)]
                        col = jnp.full((PATCH,), val, jnp.int32)
                        for il in range(CPIX):
                            vec = plsc.load_gather(
                                labq.at[par], [lane + il * PATCH, col])
                            stage[par, b, k, il] = vec
                    return _c

                lax.fori_loop(0, BS, per_b, 0)
                pending_out[par] = out_copies(q)
            for par in (0, 1):
                for cp in pending_out[par]:
                    cp.wait()
        return carry

    lax.fori_loop(0, PPW, per_patch, 0)


def _gather(train_labels, idx3):
    kern = functools.partial(
        pl.kernel,
        out_type=jax.ShapeDtypeStruct((BS, K, HW, HW), jnp.int32),
        mesh=plsc.VectorSubcoreMesh(core_axis_name="c", subcore_axis_name="s"),
        scratch_types=[
            pltpu.VMEM((2, CROWS, N_TRAIN), jnp.int32),
            pltpu.VMEM((BS, K), jnp.int32),
            pltpu.VMEM((2, BS, K, CPIX, PATCH), jnp.int32),
            pltpu.SemaphoreType.DMA,
            pltpu.SemaphoreType.DMA,
            pltpu.SemaphoreType.DMA,
        ],
        compiler_params=pltpu.CompilerParams(use_tc_tiling_on_sc=False,
                                             needs_layout_passes=False),
    )(_gather_body)
    return kern(train_labels, idx3)


def kernel(test_feature, train_features, train_labels):
    idx3, dist3 = _topk(test_feature, train_features)
    grids = _gather(train_labels, idx3)
    distances = jnp.transpose(dist3, (1, 0, 2))
    return grids, distances


# speedup vs baseline: 11.0299x; 2.1977x over previous
import functools

import jax
import jax.numpy as jnp
from jax import lax
from jax.experimental import pallas as pl
from jax.experimental.pallas import tpu as pltpu
from jax.experimental.pallas import tpu_sc as plsc

K = 20
PATCH = 16
NROWS = 14
NUM_PATCHES = NROWS * NROWS
D = 384
N_TRAIN = 500
BS = 16
HW = NROWS * PATCH

PB = 8
NSTEP = (NUM_PATCHES + PB - 1) // PB
ROWS = PB * BS

NW = 32
NPAIR = BS * NUM_PATCHES
PPW = 98
NB = 8
NGRP = (PPW + NB - 1) // NB
IDXP = 8


def _topk_body(test_ref, feat_ref, idx_ref, dist_ref):
    sims = []
    for j in range(PB):
        t = test_ref[j]
        f = feat_ref[j]
        sims.append(lax.dot_general(t, f, (((1,), (1,)), ((), ())),
                                    preferred_element_type=jnp.float32))
    sim = jnp.concatenate(sims, axis=0)

    iota = lax.broadcasted_iota(jnp.int32, (ROWS, N_TRAIN), 1)
    dists = []
    idxs = []
    for _ in range(K):
        m = jnp.max(sim, axis=1, keepdims=True)
        amin = jnp.min(jnp.where(sim == m, iota, N_TRAIN), axis=1,
                       keepdims=True)
        dists.append(m)
        idxs.append(amin)
        sim = jnp.where(iota == amin, -jnp.inf, sim)
    dist_ref[0] = jnp.concatenate(dists, axis=1)
    idx_ref[0] = jnp.concatenate(idxs, axis=1)


def _topk(test_t, feat_t):
    idxo, disto = pl.pallas_call(
        _topk_body,
        grid=(NSTEP,),
        in_specs=[
            pl.BlockSpec((PB, BS, D), lambda p: (p, 0, 0)),
            pl.BlockSpec((PB, N_TRAIN, D), lambda p: (p, 0, 0)),
        ],
        out_specs=[
            pl.BlockSpec((1, ROWS, K), lambda p: (p, 0, 0)),
            pl.BlockSpec((1, ROWS, K), lambda p: (p, 0, 0)),
        ],
        out_shape=[
            jax.ShapeDtypeStruct((NSTEP, ROWS, K), jnp.int32),
            jax.ShapeDtypeStruct((NSTEP, ROWS, K), jnp.float32),
        ],
    )(test_t, feat_t)
    idx3 = idxo.reshape(NSTEP * PB, BS, K)[:NUM_PATCHES]
    dist3 = disto.reshape(NSTEP * PB, BS, K)[:NUM_PATCHES]
    return idx3, dist3


def _gather_body(lab_hbm, idx_hbm, out_hbm, idxc, rid, stage,
                 sem_idx, sem_g, sem_o):
    wid = lax.axis_index("s") * 2 + lax.axis_index("c")
    e0 = wid * PPW
    p_lo = jnp.minimum(e0 // BS, NUM_PATCHES - IDXP)
    pltpu.sync_copy(idx_hbm.at[pl.ds(p_lo, IDXP)], idxc)

    def pair(e):
        p = e // BS
        b = e % BS
        return p, b

    def prep(t, s):
        p, b = pair(e0 + t)
        pl_ = p - p_lo
        off = p * N_TRAIN
        v0 = idxc[pl_, b, pl.ds(0, 16)] + off
        v1 = idxc[pl_, b, pl.ds(K - 16, 16)] + off
        rid[s, pl.ds(0, 16)] = v0
        rid[s, pl.ds(K - 16, 16)] = v1

    def fire_gather(t, s):
        return pltpu.make_async_copy(lab_hbm.at[rid.at[s]], stage.at[s], sem_g)

    def fire_out(t, s):
        p, b = pair(e0 + t)
        r = p // NROWS
        c = p % NROWS
        return pltpu.make_async_copy(
            stage.at[s], out_hbm.at[b, :, r, :, c, :], sem_o)

    def per_group(g, carry):
        for j in range(NB):
            t = g * NB + j

            @pl.when(jnp.logical_and(t < PPW, g > 0))
            def _wo():
                fire_out(t - NB, j).wait()

            @pl.when(t < PPW)
            def _fg():
                prep(t, j)
                fire_gather(t, j).start()
        for j in range(NB):
            t = g * NB + j

            @pl.when(t < PPW)
            def _fo():
                fire_gather(t, j).wait()
                fire_out(t, j).start()
        return carry

    lax.fori_loop(0, NGRP, per_group, 0)
    for t in range(PPW - NB, PPW):
        fire_out(jnp.int32(t), t % NB).wait()


def _gather(lab3, idx3):
    kern = functools.partial(
        pl.kernel,
        out_type=jax.ShapeDtypeStruct((BS, K, NROWS, PATCH, NROWS, PATCH),
                                      jnp.int32),
        mesh=plsc.VectorSubcoreMesh(core_axis_name="c", subcore_axis_name="s"),
        scratch_types=[
            pltpu.VMEM((IDXP, BS, K), jnp.int32),
            pltpu.VMEM((NB, K), jnp.int32),
            pltpu.VMEM((NB, K, PATCH, PATCH), jnp.int32),
            pltpu.SemaphoreType.DMA,
            pltpu.SemaphoreType.DMA,
            pltpu.SemaphoreType.DMA,
        ],
        compiler_params=pltpu.CompilerParams(use_tc_tiling_on_sc=False,
                                             needs_layout_passes=False),
    )(_gather_body)
    return kern(lab3, idx3)


def kernel(test_feature, train_features, train_labels):
    test_t = jnp.transpose(test_feature, (1, 0, 2))
    feat_t = jnp.transpose(train_features, (0, 2, 1))
    lab3 = jnp.transpose(train_labels, (0, 2, 1)).reshape(
        NUM_PATCHES * N_TRAIN, PATCH, PATCH)

    idx3, dist3 = _topk(test_t, feat_t)
    out6 = _gather(lab3, idx3)
    grids = out6.reshape(BS, K, HW, HW)
    distances = jnp.transpose(dist3, (1, 0, 2))
    return grids, distances
